# Initial kernel scaffold; baseline (speedup 1.0000x reference)
import jax
import jax.numpy as jnp
from jax.experimental import pallas as pl


def kernel(x, w, gamma, beta):
    raise NotImplementedError("write your pallas kernel here")


# speedup vs baseline: 1.0415x; 1.0415x over previous
import functools

import jax
import jax.numpy as jnp
from jax import lax
from jax.experimental import pallas as pl
from jax.experimental.pallas import tpu as pltpu

_VMEM_LIMIT = 48 * 1024 * 1024


def _conv_stats_kernel(x_ref, w_ref, mask_ref, y_ref, psum_ref, pssq_ref, rhs_ref,
                       *, H, W, Cin):
    d = pl.program_id(1)
    HW = H * W
    Pw = W + 1
    K3 = 3 * Cin

    t = 0
    for dh in (-1, 0, 1):
        for dw in (-1, 0, 1):
            off = Pw + dh * W + dw
            slab = x_ref[0, pl.ds(d, 3), :, pl.ds(off, HW)]
            slab = slab.reshape(K3, HW)
            if not (dh == 0 and dw == 0):
                slab = slab * mask_ref[t:t + 1, :]
            rhs_ref[t * K3:(t + 1) * K3, :] = slab
            t += 1

    acc = jnp.dot(w_ref[...], rhs_ref[...], preferred_element_type=jnp.float32)
    y_ref[0, 0] = acc.astype(jnp.bfloat16)

    ps = jnp.sum(acc, axis=1, keepdims=True)
    ss = jnp.sum(acc * acc, axis=1, keepdims=True)

    @pl.when(d == 0)
    def _():
        psum_ref[0] = ps
        pssq_ref[0] = ss

    @pl.when(d > 0)
    def _():
        psum_ref[0] += ps
        pssq_ref[0] += ss


def _bn_lrelu_kernel(y_ref, scale_ref, shift_ref, o_ref, *, G, slope):
    for g in range(G):
        z = y_ref[0, g].astype(jnp.float32) * scale_ref[...] + shift_ref[...]
        o_ref[0, :, g, :] = jnp.where(z > 0, z, slope * z)


@functools.partial(jax.jit, static_argnames=("eps", "slope"))
def _conv_bn_lrelu(x, w, gamma, beta, *, eps=1e-5, slope=0.01):
    N, Cin, D, H, W = x.shape
    Cout = w.shape[0]
    HW = H * W
    Pw = W + 1
    Hw_p = HW + 2 * Pw
    K = 27 * Cin

    x_t = jnp.transpose(x, (0, 2, 1, 3, 4)).reshape(N, D, Cin, HW).astype(jnp.bfloat16)
    x_p = jnp.pad(x_t, ((0, 0), (1, 1), (0, 0), (Pw, Pw)))

    w_l = jnp.transpose(w, (0, 3, 4, 2, 1)).reshape(Cout, K).astype(jnp.bfloat16)

    hh = jnp.arange(H, dtype=jnp.int32).reshape(H, 1)
    ww = jnp.arange(W, dtype=jnp.int32).reshape(1, W)
    rows = []
    for dh in (-1, 0, 1):
        for dw in (-1, 0, 1):
            ok = (hh + dh >= 0) & (hh + dh < H) & (ww + dw >= 0) & (ww + dw < W)
            rows.append(ok.reshape(HW))
    mask9 = jnp.stack(rows, axis=0).astype(jnp.bfloat16)

    kern1 = functools.partial(_conv_stats_kernel, H=H, W=W, Cin=Cin)
    y1, psum, pssq = pl.pallas_call(
        kern1,
        grid=(N, D),
        in_specs=[
            pl.BlockSpec((1, D + 2, Cin, Hw_p), lambda n, d: (n, 0, 0, 0)),
            pl.BlockSpec((Cout, K), lambda n, d: (0, 0)),
            pl.BlockSpec((9, HW), lambda n, d: (0, 0)),
        ],
        out_specs=[
            pl.BlockSpec((1, 1, Cout, HW), lambda n, d: (n, d, 0, 0)),
            pl.BlockSpec((1, Cout, 1), lambda n, d: (n, 0, 0)),
            pl.BlockSpec((1, Cout, 1), lambda n, d: (n, 0, 0)),
        ],
        out_shape=(
            jax.ShapeDtypeStruct((N, D, Cout, HW), jnp.bfloat16),
            jax.ShapeDtypeStruct((N, Cout, 1), jnp.float32),
            jax.ShapeDtypeStruct((N, Cout, 1), jnp.float32),
        ),
        scratch_shapes=[pltpu.VMEM((K, HW), jnp.bfloat16)],
        compiler_params=pltpu.CompilerParams(
            dimension_semantics=("parallel", "arbitrary"),
            vmem_limit_bytes=_VMEM_LIMIT),
    )(x_p, w_l, mask9)

    count = float(N * D * HW)
    g32 = gamma.astype(jnp.float32)
    b32 = beta.astype(jnp.float32)
    mean = jnp.sum(psum[:, :, 0], axis=0) / count
    ex2 = jnp.sum(pssq[:, :, 0], axis=0) / count
    var = jnp.maximum(ex2 - mean * mean, 0.0)
    inv_std = lax.rsqrt(var + eps)
    scale = (g32 * inv_std).reshape(Cout, 1)
    shift = (b32 - mean * g32 * inv_std).reshape(Cout, 1)

    G = 8 if D % 8 == 0 else D
    kern2 = functools.partial(_bn_lrelu_kernel, G=G, slope=slope)
    out4 = pl.pallas_call(
        kern2,
        grid=(N, D // G),
        in_specs=[
            pl.BlockSpec((1, G, Cout, HW), lambda n, g: (n, g, 0, 0)),
            pl.BlockSpec((Cout, 1), lambda n, g: (0, 0)),
            pl.BlockSpec((Cout, 1), lambda n, g: (0, 0)),
        ],
        out_specs=pl.BlockSpec((1, Cout, G, HW), lambda n, g: (n, 0, g, 0)),
        out_shape=jax.ShapeDtypeStruct((N, Cout, D, HW), jnp.float32),
        compiler_params=pltpu.CompilerParams(
            dimension_semantics=("parallel", "parallel"),
            vmem_limit_bytes=_VMEM_LIMIT),
    )(y1, scale, shift)

    return out4.reshape(N, Cout, D, H, W)


def kernel(x, w, gamma, beta):
    return _conv_bn_lrelu(x, w, gamma, beta)


# speedup vs baseline: 1.4868x; 1.4275x over previous
import functools

import jax
import jax.numpy as jnp
from jax import lax
from jax.experimental import pallas as pl
from jax.experimental.pallas import tpu as pltpu

_VMEM_LIMIT = 48 * 1024 * 1024
_TAPS = tuple((dh, dw) for dh in (-1, 0, 1) for dw in (-1, 0, 1))


def _conv_stats_kernel(x_ref, w_ref, mask_ref, y_ref, psum_ref, pssq_ref, rhs_ref,
                       *, H, W, Cin, D, Cout):
    HW = H * W
    Pw = W + 1
    B = 9 * Cin

    for p in range(D):
        for t, (dh, dw) in enumerate(_TAPS):
            off = Pw + dh * W + dw
            slab = x_ref[0, p, :, off:off + HW]
            if not (dh == 0 and dw == 0):
                slab = slab * mask_ref[t:t + 1, :]
            rhs_ref[p * B + t * Cin:p * B + (t + 1) * Cin, :] = slab

    ps = jnp.zeros((Cout, 1), jnp.float32)
    ss = jnp.zeros((Cout, 1), jnp.float32)
    for d in range(D):
        qlo = max(d - 1, 0)
        qhi = min(d + 1, D - 1)
        c0 = (qlo - (d - 1)) * B
        c1 = (qhi - (d - 1) + 1) * B
        acc = jnp.dot(w_ref[:, c0:c1], rhs_ref[qlo * B:(qhi + 1) * B, :],
                      preferred_element_type=jnp.float32)
        y_ref[0, d] = acc.astype(jnp.bfloat16)
        ps = ps + jnp.sum(acc, axis=1, keepdims=True)
        ss = ss + jnp.sum(acc * acc, axis=1, keepdims=True)
    psum_ref[0] = ps
    pssq_ref[0] = ss


def _bn_lrelu_kernel(y_ref, scale_ref, shift_ref, o_ref, *, G, slope):
    for g in range(G):
        z = y_ref[0, g].astype(jnp.float32) * scale_ref[...] + shift_ref[...]
        o_ref[0, :, g, :] = jnp.where(z > 0, z, slope * z)


@functools.partial(jax.jit, static_argnames=("eps", "slope"))
def _conv_bn_lrelu(x, w, gamma, beta, *, eps=1e-5, slope=0.01):
    N, Cin, D, H, W = x.shape
    Cout = w.shape[0]
    HW = H * W
    Pw = W + 1
    Hw_p = HW + 2 * Pw
    K = 27 * Cin

    x_t = jnp.transpose(x, (0, 2, 1, 3, 4)).reshape(N, D, Cin, HW).astype(jnp.bfloat16)
    x_p = jnp.pad(x_t, ((0, 0), (0, 0), (0, 0), (Pw, Pw)))

    w_l = jnp.transpose(w, (0, 2, 3, 4, 1)).reshape(Cout, K).astype(jnp.bfloat16)

    hh = jnp.arange(H, dtype=jnp.int32).reshape(H, 1)
    ww = jnp.arange(W, dtype=jnp.int32).reshape(1, W)
    rows = []
    for dh, dw in _TAPS:
        ok = (hh + dh >= 0) & (hh + dh < H) & (ww + dw >= 0) & (ww + dw < W)
        rows.append(ok.reshape(HW))
    mask9 = jnp.stack(rows, axis=0).astype(jnp.bfloat16)

    kern1 = functools.partial(_conv_stats_kernel, H=H, W=W, Cin=Cin, D=D, Cout=Cout)
    y1, psum, pssq = pl.pallas_call(
        kern1,
        grid=(N,),
        in_specs=[
            pl.BlockSpec((1, D, Cin, Hw_p), lambda n: (n, 0, 0, 0)),
            pl.BlockSpec((Cout, K), lambda n: (0, 0)),
            pl.BlockSpec((9, HW), lambda n: (0, 0)),
        ],
        out_specs=[
            pl.BlockSpec((1, D, Cout, HW), lambda n: (n, 0, 0, 0)),
            pl.BlockSpec((1, Cout, 1), lambda n: (n, 0, 0)),
            pl.BlockSpec((1, Cout, 1), lambda n: (n, 0, 0)),
        ],
        out_shape=(
            jax.ShapeDtypeStruct((N, D, Cout, HW), jnp.bfloat16),
            jax.ShapeDtypeStruct((N, Cout, 1), jnp.float32),
            jax.ShapeDtypeStruct((N, Cout, 1), jnp.float32),
        ),
        scratch_shapes=[pltpu.VMEM((D * 9 * Cin, HW), jnp.bfloat16)],
        compiler_params=pltpu.CompilerParams(
            dimension_semantics=("parallel",),
            vmem_limit_bytes=_VMEM_LIMIT),
    )(x_p, w_l, mask9)

    count = float(N * D * HW)
    g32 = gamma.astype(jnp.float32)
    b32 = beta.astype(jnp.float32)
    mean = jnp.sum(psum[:, :, 0], axis=0) / count
    ex2 = jnp.sum(pssq[:, :, 0], axis=0) / count
    var = jnp.maximum(ex2 - mean * mean, 0.0)
    inv_std = lax.rsqrt(var + eps)
    scale = (g32 * inv_std).reshape(Cout, 1)
    shift = (b32 - mean * g32 * inv_std).reshape(Cout, 1)

    kern2 = functools.partial(_bn_lrelu_kernel, G=D, slope=slope)
    out4 = pl.pallas_call(
        kern2,
        grid=(N,),
        in_specs=[
            pl.BlockSpec((1, D, Cout, HW), lambda n: (n, 0, 0, 0)),
            pl.BlockSpec((Cout, 1), lambda n: (0, 0)),
            pl.BlockSpec((Cout, 1), lambda n: (0, 0)),
        ],
        out_specs=pl.BlockSpec((1, Cout, D, HW), lambda n: (n, 0, 0, 0)),
        out_shape=jax.ShapeDtypeStruct((N, Cout, D, HW), jnp.float32),
        compiler_params=pltpu.CompilerParams(
            dimension_semantics=("parallel",),
            vmem_limit_bytes=_VMEM_LIMIT),
    )(y1, scale, shift)

    return out4.reshape(N, Cout, D, H, W)


def kernel(x, w, gamma, beta):
    return _conv_bn_lrelu(x, w, gamma, beta)
